# Initial kernel scaffold; baseline (speedup 1.0000x reference)
import jax
import jax.numpy as jnp
from jax.experimental import pallas as pl


def kernel(inputs, adj_mx, forward_index, Wg0, bg0, Wc0, bc0, Wg1, bg1, Wc1, bc1, Wp, bp):
    raise NotImplementedError("write your pallas kernel here")


# speedup vs baseline: 2.9439x; 2.9439x over previous
import functools

import jax
import jax.numpy as jnp
from jax.experimental import pallas as pl

N = 3960
U = 64
B = 4
R = 360
NB = N // R


def _dot(a, b):
    return jax.lax.dot_general(a, b, (((1,), (0,)), ((), ())),
                               preferred_element_type=jnp.float32)


def _dot_t(a, b):
    return jax.lax.dot_general(a, b, (((0,), (0,)), ((), ())),
                               preferred_element_type=jnp.float32)


def _inv(d):
    return jnp.where(d > 0, 1.0 / d, 0.0)


def _p1_body(a_ref, xe_full_ref, xe_blk_ref, sf_ref, sb_ref):
    i = pl.program_id(0)
    a = a_ref[...]
    sf_ref[...] = _dot(a, xe_full_ref[...])
    bwd = _dot_t(a, xe_blk_ref[...])

    @pl.when(i == 0)
    def _():
        sb_ref[...] = bwd

    @pl.when(i != 0)
    def _():
        sb_ref[...] += bwd


def _p2_body(a_ref, sfe_full_ref, sfe_blk_ref, sbe_blk_ref, x_blk_ref,
             t2f_ref, t2b_ref):
    i = pl.program_id(0)
    a = a_ref[...]
    sfe = sfe_full_ref[...]
    t1f_full = sfe[:, :4] * _inv(sfe[:, 4:5])
    dinv_blk = _inv(sfe_blk_ref[:, 4:5])
    t2f_ref[...] = 2.0 * dinv_blk * _dot(a, t1f_full) - x_blk_ref[...]
    sbe_blk = sbe_blk_ref[...]
    t1b_blk = sbe_blk[:, :4] * _inv(sbe_blk[:, 4:5])
    bwd = _dot_t(a, t1b_blk)

    @pl.when(i == 0)
    def _():
        t2b_ref[...] = bwd

    @pl.when(i != 0)
    def _():
        t2b_ref[...] += bwd


def _pw0_body(x_ref, sfe_ref, t2f_ref, sbe_ref, t2braw_ref,
              w5u_ref, w5c_ref, bgu_ref, bc_ref, h1_ref):
    x = x_ref[...]
    sfe = sfe_ref[...]
    sbe = sbe_ref[...]
    dinv = _inv(sfe[:, 4:5])
    dcinv = _inv(sbe[:, 4:5])
    t1f = sfe[:, :4] * dinv
    t1b = sbe[:, :4] * dcinv
    t2f = t2f_ref[...]
    t2b = 2.0 * dcinv * t2braw_ref[...] - x
    terms = (x, t1f, t2f, t1b, t2b)
    for b in range(B):
        accu = jnp.broadcast_to(bgu_ref[...], (R, U))
        accc = jnp.broadcast_to(bc_ref[...], (R, U))
        for m, t in enumerate(terms):
            col = t[:, b:b + 1]
            accu = accu + col * w5u_ref[m:m + 1, :]
            accc = accc + col * w5c_ref[m:m + 1, :]
        u = jax.nn.sigmoid(accu)
        c = jnp.tanh(accc)
        h1_ref[:, b, :] = (1.0 - u) * c


def _p3_body(a_ref, h_full_ref, h_blk_ref, sfe_blk_ref, t1f_ref, s1b_ref):
    i = pl.program_id(0)
    a = a_ref[...]
    dinv_blk = _inv(sfe_blk_ref[:, 4:5])
    t1f_ref[...] = dinv_blk * _dot(a, h_full_ref[...])
    bwd = _dot_t(a, h_blk_ref[...])

    @pl.when(i == 0)
    def _():
        s1b_ref[...] = bwd

    @pl.when(i != 0)
    def _():
        s1b_ref[...] += bwd


def _p4_body(a_ref, t1f_full_ref, h_blk_ref, sfe_blk_ref, s1b_blk_ref,
             sbe_blk_ref, t2f_ref, s2b_ref):
    i = pl.program_id(0)
    a = a_ref[...]
    dinv_blk = _inv(sfe_blk_ref[:, 4:5])
    t2f_ref[...] = 2.0 * dinv_blk * _dot(a, t1f_full_ref[...]) - h_blk_ref[...]
    t1b_blk = _inv(sbe_blk_ref[:, 4:5]) * s1b_blk_ref[...]
    bwd = _dot_t(a, t1b_blk)

    @pl.when(i == 0)
    def _():
        s2b_ref[...] = bwd

    @pl.when(i != 0)
    def _():
        s2b_ref[...] += bwd


def _pw1_body(h_ref, t1f_ref, t2f_ref, s1b_ref, s2b_ref, sbe_ref,
              wgu_ref, wc_ref, bgu_ref, bc_ref, wp_ref, bp_ref,
              h2_ref, proj_ref):
    dcinv = _inv(sbe_ref[:, 4:5])
    h = h_ref[...]
    t1f = t1f_ref[...]
    t2f = t2f_ref[...]
    t1b = dcinv * s1b_ref[...]
    t2b = 2.0 * dcinv * s2b_ref[...] - h
    wgu = wgu_ref[...]
    wc = wc_ref[...]
    cols = []
    for b in range(B):
        sl = slice(b * U, (b + 1) * U)
        xb = jnp.concatenate(
            [h[:, sl], t1f[:, sl], t2f[:, sl], t1b[:, sl], t2b[:, sl]],
            axis=1)
        u = jax.nn.sigmoid(_dot(xb, wgu) + bgu_ref[...])
        c = jnp.tanh(_dot(xb, wc) + bc_ref[...])
        h2b = (1.0 - u) * c
        h2_ref[:, b, :] = h2b
        cols.append(jnp.sum(h2b * wp_ref[...], axis=1, keepdims=True))
    proj_ref[...] = jnp.concatenate(cols, axis=1) + bp_ref[0, 0]


def _blk(shape):
    return pl.BlockSpec(shape, lambda i: (i,) + (0,) * (len(shape) - 1))


def _full(shape):
    return pl.BlockSpec(shape, lambda i: (0,) * len(shape))


@functools.partial(jax.jit, static_argnames=())
def _run(x, adj, Wg0, bg0, Wc0, bc0, Wg1, bg1, Wc1, bc1, Wp, bp):
    f32 = jnp.float32
    xe = jnp.concatenate([x, jnp.ones((N, 1), f32)], axis=1)

    idx0 = jnp.arange(5) * (1 + U)
    W5u = Wg0[idx0][:, U:]
    W5c = Wc0[idx0]
    idx1 = (jnp.arange(5)[:, None] * (2 * U) + jnp.arange(U)[None, :]).reshape(-1)
    Wg1u = Wg1[idx1][:, U:]
    Wc1f = Wc1[idx1]
    bg0u = bg0[U:].reshape(1, U)
    bc0r = bc0.reshape(1, U)
    bg1u = bg1[U:].reshape(1, U)
    bc1r = bc1.reshape(1, U)
    wp_t = Wp.reshape(1, U)
    bp_r = bp.reshape(1, 1)

    grid = (NB,)
    a_spec = _blk((R, N))

    sf_e, sb_e = pl.pallas_call(
        _p1_body,
        grid=grid,
        in_specs=[a_spec, _full((N, 5)), _blk((R, 5))],
        out_specs=[_blk((R, 5)), _full((N, 5))],
        out_shape=[jax.ShapeDtypeStruct((N, 5), f32),
                   jax.ShapeDtypeStruct((N, 5), f32)],
    )(adj, xe, xe)

    t2f0, t2b0raw = pl.pallas_call(
        _p2_body,
        grid=grid,
        in_specs=[a_spec, _full((N, 5)), _blk((R, 5)), _blk((R, 5)),
                  _blk((R, 4))],
        out_specs=[_blk((R, 4)), _full((N, 4))],
        out_shape=[jax.ShapeDtypeStruct((N, 4), f32),
                   jax.ShapeDtypeStruct((N, 4), f32)],
    )(adj, sf_e, sf_e, sb_e, x)

    h1_3d = pl.pallas_call(
        _pw0_body,
        grid=grid,
        in_specs=[_blk((R, 4)), _blk((R, 5)), _blk((R, 4)), _blk((R, 5)),
                  _blk((R, 4)), _full((5, U)), _full((5, U)),
                  _full((1, U)), _full((1, U))],
        out_specs=[_blk((R, B, U))],
        out_shape=[jax.ShapeDtypeStruct((N, B, U), f32)],
    )(x, sf_e, t2f0, sb_e, t2b0raw, W5u, W5c, bg0u, bc0r)[0]

    H = h1_3d.reshape(N, B * U)

    t1f1, s1b = pl.pallas_call(
        _p3_body,
        grid=grid,
        in_specs=[a_spec, _full((N, B * U)), _blk((R, B * U)), _blk((R, 5))],
        out_specs=[_blk((R, B * U)), _full((N, B * U))],
        out_shape=[jax.ShapeDtypeStruct((N, B * U), f32),
                   jax.ShapeDtypeStruct((N, B * U), f32)],
    )(adj, H, H, sf_e)

    t2f1, s2b = pl.pallas_call(
        _p4_body,
        grid=grid,
        in_specs=[a_spec, _full((N, B * U)), _blk((R, B * U)), _blk((R, 5)),
                  _blk((R, B * U)), _blk((R, 5))],
        out_specs=[_blk((R, B * U)), _full((N, B * U))],
        out_shape=[jax.ShapeDtypeStruct((N, B * U), f32),
                   jax.ShapeDtypeStruct((N, B * U), f32)],
    )(adj, t1f1, H, sf_e, s1b, sb_e)

    h2_3d, proj = pl.pallas_call(
        _pw1_body,
        grid=grid,
        in_specs=[_blk((R, B * U))] * 5 + [_blk((R, 5)),
                  _full((5 * U, U)), _full((5 * U, U)),
                  _full((1, U)), _full((1, U)), _full((1, U)),
                  _full((1, 1))],
        out_specs=[_blk((R, B, U)), _blk((R, B))],
        out_shape=[jax.ShapeDtypeStruct((N, B, U), f32),
                   jax.ShapeDtypeStruct((N, B), f32)],
    )(H, t1f1, t2f1, s1b, s2b, sb_e, Wg1u, Wc1f, bg1u, bc1r, wp_t, bp_r)

    output = proj.T
    h1_flat = h1_3d.transpose(1, 0, 2).reshape(B, N * U)
    h2_flat = h2_3d.transpose(1, 0, 2).reshape(B, N * U)
    hidden = jnp.stack([h1_flat, h2_flat])
    return output, hidden


def kernel(inputs, adj_mx, forward_index, Wg0, bg0, Wc0, bc0,
           Wg1, bg1, Wc1, bc1, Wp, bp):
    x = inputs.T.astype(jnp.float32)
    return _run(x, adj_mx, Wg0, bg0, Wc0, bc0, Wg1, bg1, Wc1, bc1, Wp, bp)
